# baseline (speedup 1.0000x reference)
import functools
import jax
import jax.numpy as jnp
from jax import lax
from jax.experimental import pallas as pl
from jax.experimental.pallas import tpu as pltpu
from jax.experimental.pallas import tpu_sc as plsc

MODEL_DIM = 4096
PROMPT_DIM = 64
NUM_EXPERTS = 64
TM = 1024

NC = 2
NS = 16
L = 16
NW = NC * NS
CT = 128
NG = CT // L


def _logits_kernel(x_ref, p_ref, wi_ref, wp_ref, b_ref, lg_ref):
    lg_ref[...] = (
        jnp.dot(x_ref[...], wi_ref[...], preferred_element_type=jnp.float32)
        + jnp.dot(p_ref[...], wp_ref[...], preferred_element_type=jnp.float32)
        + b_ref[...])


def _tc_logits(x, prompt, wi, wp, b2):
    T = x.shape[0]
    return pl.pallas_call(
        _logits_kernel,
        grid=(T // TM,),
        in_specs=[
            pl.BlockSpec((TM, MODEL_DIM), lambda i: (i, 0)),
            pl.BlockSpec((TM, PROMPT_DIM), lambda i: (i, 0)),
            pl.BlockSpec((MODEL_DIM, NUM_EXPERTS), lambda i: (0, 0)),
            pl.BlockSpec((PROMPT_DIM, NUM_EXPERTS), lambda i: (0, 0)),
            pl.BlockSpec((1, NUM_EXPERTS), lambda i: (0, 0)),
        ],
        out_specs=pl.BlockSpec((TM, NUM_EXPERTS), lambda i: (i, 0)),
        out_shape=jax.ShapeDtypeStruct((T, NUM_EXPERTS), jnp.float32),
    )(x, prompt, wi, wp, b2)


def _gate_sc_body(TW, lg_hbm, zb_hbm, m0_hbm, m1_hbm, g0_hbm, g1_hbm,
                  lg_v, m0v, m1v, g0c, g1c, i1c, i2c):
    wid = lax.axis_index("s") * NC + lax.axis_index("c")
    base = wid * TW
    pltpu.sync_copy(lg_hbm.at[pl.ds(base * NUM_EXPERTS, TW * NUM_EXPERTS)],
                    lg_v)
    pltpu.sync_copy(zb_hbm, m0v)
    pltpu.sync_copy(zb_hbm, m1v)

    iota16 = jnp.arange(L, dtype=jnp.int32)
    ones16 = jnp.ones((L,), jnp.int32)
    zeros16 = jnp.zeros((L,), jnp.int32)

    def chunk_body(c, carry):
        for g in range(NG):
            rbase = (c * CT + g * L + iota16) * NUM_EXPERTS
            lbase = (g * L + iota16) * NUM_EXPERTS
            t1 = plsc.load_gather(lg_v, [rbase])
            i1 = jnp.zeros((L,), jnp.int32)
            t2 = jnp.full((L,), -jnp.inf, jnp.float32)
            i2 = jnp.zeros((L,), jnp.int32)
            for e in range(1, NUM_EXPERTS):
                ev = jnp.full((L,), e, jnp.int32)
                v = plsc.load_gather(lg_v, [rbase + e])
                gt1 = v > t1
                gt2 = v > t2
                t2 = jnp.where(gt2, jnp.where(gt1, t1, v), t2)
                i2 = jnp.where(gt2, jnp.where(gt1, i1, ev), i2)
                t1 = jnp.where(gt1, v, t1)
                i1 = jnp.where(gt1, ev, i1)
            plsc.store_scatter(m0v, [lbase + i1], ones16)
            plsc.store_scatter(m1v, [lbase + i2], ones16)
            a = jnp.exp(t2 - t1)
            g0 = 1.0 / (1.0 + a)
            g0c[pl.ds(g * L, L)] = g0
            g1c[pl.ds(g * L, L)] = 1.0 - g0
            i1c[pl.ds(g * L, L)] = i1
            i2c[pl.ds(g * L, L)] = i2
        ob = base + c * CT
        pltpu.sync_copy(m0v, m0_hbm.at[pl.ds(ob * NUM_EXPERTS,
                                             CT * NUM_EXPERTS)])
        pltpu.sync_copy(m1v, m1_hbm.at[pl.ds(ob * NUM_EXPERTS,
                                             CT * NUM_EXPERTS)])
        pltpu.sync_copy(g0c, g0_hbm.at[pl.ds(ob, CT)])
        pltpu.sync_copy(g1c, g1_hbm.at[pl.ds(ob, CT)])
        for g in range(NG):
            lbase = (g * L + iota16) * NUM_EXPERTS
            plsc.store_scatter(m0v, [lbase + i1c[pl.ds(g * L, L)]], zeros16)
            plsc.store_scatter(m1v, [lbase + i2c[pl.ds(g * L, L)]], zeros16)
        return carry

    lax.fori_loop(0, TW // CT, chunk_body, 0)


def _sc_gate(logits):
    T = logits.shape[0]
    TW = T // NW
    lg_flat = logits.reshape(T * NUM_EXPERTS)
    zb = jnp.zeros((CT * NUM_EXPERTS,), jnp.int32)
    mesh = plsc.VectorSubcoreMesh(core_axis_name="c", subcore_axis_name="s")
    k = pl.kernel(
        functools.partial(_gate_sc_body, TW),
        out_type=(
            jax.ShapeDtypeStruct((T * NUM_EXPERTS,), jnp.int32),
            jax.ShapeDtypeStruct((T * NUM_EXPERTS,), jnp.int32),
            jax.ShapeDtypeStruct((T,), jnp.float32),
            jax.ShapeDtypeStruct((T,), jnp.float32),
        ),
        mesh=mesh,
        compiler_params=pltpu.CompilerParams(needs_layout_passes=False),
        scratch_types=[
            pltpu.VMEM((TW * NUM_EXPERTS,), jnp.float32),
            pltpu.VMEM((CT * NUM_EXPERTS,), jnp.int32),
            pltpu.VMEM((CT * NUM_EXPERTS,), jnp.int32),
            pltpu.VMEM((CT,), jnp.float32),
            pltpu.VMEM((CT,), jnp.float32),
            pltpu.VMEM((CT,), jnp.int32),
            pltpu.VMEM((CT,), jnp.int32),
        ],
    )
    m0f, m1f, g0, g1 = k(lg_flat, zb)
    return (m0f.reshape(T, NUM_EXPERTS), m1f.reshape(T, NUM_EXPERTS), g0, g1)


def kernel(input, prompt, W, b):
    x = input.astype(jnp.float32)
    wi = W[:, :MODEL_DIM].T
    wp = W[:, MODEL_DIM:].T
    b2 = b.reshape(1, NUM_EXPERTS)
    logits = _tc_logits(x, prompt, wi, wp, b2)
    return _sc_gate(logits)


# speedup vs baseline: 1.3423x; 1.3423x over previous
import jax
import jax.numpy as jnp
from jax.experimental import pallas as pl

MODEL_DIM = 4096
PROMPT_DIM = 64
NUM_EXPERTS = 64
TM = 1024


def _router_kernel(x_ref, p_ref, wi_ref, wp_ref, b_ref,
                   m0_ref, m1_ref, g0_ref, g1_ref):
    logits = (jnp.dot(x_ref[...], wi_ref[...], preferred_element_type=jnp.float32)
              + jnp.dot(p_ref[...], wp_ref[...], preferred_element_type=jnp.float32)
              + b_ref[...])

    iota = jax.lax.broadcasted_iota(jnp.int32, logits.shape, 1)
    top1 = jnp.max(logits, axis=1, keepdims=True)
    idx1 = jnp.min(jnp.where(logits == top1, iota, NUM_EXPERTS),
                   axis=1, keepdims=True)
    mask0 = iota == idx1
    rest = jnp.where(mask0, -jnp.inf, logits)
    top2 = jnp.max(rest, axis=1, keepdims=True)
    idx2 = jnp.min(jnp.where(rest == top2, iota, NUM_EXPERTS),
                   axis=1, keepdims=True)
    mask1 = iota == idx2

    e = jnp.exp(logits - top1)
    s = jnp.sum(e, axis=1)
    gs0 = jnp.sum(jnp.where(mask0, e, 0.0), axis=1) / s
    gs1 = jnp.sum(jnp.where(mask1, e, 0.0), axis=1) / s
    denom = jnp.maximum(gs0 + gs1, jnp.finfo(jnp.float32).eps)

    m0_ref[...] = mask0.astype(jnp.int32)
    m1_ref[...] = mask1.astype(jnp.int32)
    g0_ref[...] = gs0 / denom
    g1_ref[...] = gs1 / denom


def kernel(input, prompt, W, b):
    T = input.shape[0]
    x = input.astype(jnp.float32)
    wi = W[:, :MODEL_DIM].T
    wp = W[:, MODEL_DIM:].T
    b2 = b.reshape(1, NUM_EXPERTS)

    grid = (T // TM,)
    out_shape = (
        jax.ShapeDtypeStruct((T, NUM_EXPERTS), jnp.int32),
        jax.ShapeDtypeStruct((T, NUM_EXPERTS), jnp.int32),
        jax.ShapeDtypeStruct((T,), jnp.float32),
        jax.ShapeDtypeStruct((T,), jnp.float32),
    )
    in_specs = [
        pl.BlockSpec((TM, MODEL_DIM), lambda i: (i, 0)),
        pl.BlockSpec((TM, PROMPT_DIM), lambda i: (i, 0)),
        pl.BlockSpec((MODEL_DIM, NUM_EXPERTS), lambda i: (0, 0)),
        pl.BlockSpec((PROMPT_DIM, NUM_EXPERTS), lambda i: (0, 0)),
        pl.BlockSpec((1, NUM_EXPERTS), lambda i: (0, 0)),
    ]
    out_specs = (
        pl.BlockSpec((TM, NUM_EXPERTS), lambda i: (i, 0)),
        pl.BlockSpec((TM, NUM_EXPERTS), lambda i: (i, 0)),
        pl.BlockSpec((TM,), lambda i: (i,)),
        pl.BlockSpec((TM,), lambda i: (i,)),
    )
    return pl.pallas_call(
        _router_kernel,
        grid=grid,
        in_specs=in_specs,
        out_specs=out_specs,
        out_shape=out_shape,
    )(x, prompt, wi, wp, b2)
